# baseline (speedup 1.0000x reference)
import functools

import jax
import jax.numpy as jnp
from jax import lax
from jax.experimental import pallas as pl
from jax.experimental.pallas import tpu as pltpu
from jax.experimental.pallas import tpu_sc as plsc

B, T, C, D, H = 16, 2048, 1024, 64, 1024

_NW = 32
_HPW = H // _NW
_HC = 16
_NHC = _HPW // _HC
_L = 16


def _project_body(cb_ref, w_ref, bias_ref, pt_ref):
    pt = lax.dot_general(
        w_ref[...], cb_ref[...], (((0,), (1,)), ((), ())),
        preferred_element_type=jnp.float32, precision=lax.Precision.HIGHEST)
    pt_ref[...] = pt + bias_ref[...]


def _project(codebook, W, bias):
    return pl.pallas_call(
        _project_body,
        out_shape=jax.ShapeDtypeStruct((H, C), jnp.float32),
    )(codebook, W, bias.reshape(H, 1))


_sc_mesh = plsc.VectorSubcoreMesh(core_axis_name="c", subcore_axis_name="s")


@functools.partial(
    pl.kernel,
    out_type=jax.ShapeDtypeStruct((B * H, T), jnp.float32),
    mesh=_sc_mesh,
    scratch_types=[
        pltpu.VMEM((_HPW * C,), jnp.float32),
        pltpu.VMEM((2, T), jnp.int32),
        pltpu.VMEM((2, _HC, T), jnp.float32),
        pltpu.SemaphoreType.DMA,
        pltpu.SemaphoreType.DMA,
        pltpu.SemaphoreType.DMA,
    ],
    compiler_params=pltpu.CompilerParams(
        use_tc_tiling_on_sc=False, needs_layout_passes=False),
)
def _gather_kernel(pt_hbm, idx_hbm, out_hbm, rows, idxb, outb, osem0, osem1, isem):
    cid = lax.axis_index("c")
    sid = lax.axis_index("s")
    w = sid * 2 + cid
    hb = w * _HPW

    pltpu.sync_copy(pt_hbm.at[pl.ds(hb * C, _HPW * C)], rows)
    pltpu.sync_copy(idx_hbm.at[pl.ds(0, T)], idxb.at[0])

    osems = (osem0, osem1)
    ocopy = [None, None]
    icopy = None
    for bb in range(B):
        pb = bb % 2
        if bb + 1 < B:
            icopy = pltpu.async_copy(
                idx_hbm.at[pl.ds((bb + 1) * T, T)], idxb.at[(bb + 1) % 2], isem)
        for hc in range(_NHC):
            p = (bb * _NHC + hc) % 2
            if ocopy[p] is not None:
                ocopy[p].wait()

            def t_body(t, _, pb=pb, p=p, hc=hc):
                base = t * _L
                idxv = idxb[pb, pl.ds(base, _L)]
                for hh in range(_HC):
                    vals = plsc.load_gather(rows, [idxv + (hc * _HC + hh) * C])
                    outb[p, hh, pl.ds(base, _L)] = vals
                return 0

            lax.fori_loop(0, T // _L, t_body, 0)
            ocopy[p] = pltpu.async_copy(
                outb.at[p], out_hbm.at[pl.ds(bb * H + hb + hc * _HC, _HC)],
                osems[p])
        if bb + 1 < B:
            icopy.wait()
    ocopy[0].wait()
    ocopy[1].wait()


def kernel(indices, codebook, W, b):
    pt = _project(codebook, W, b)
    idx_flat = indices.astype(jnp.int32).reshape(B * T)
    out = _gather_kernel(pt.reshape(H * C), idx_flat)
    return out.reshape(B, H, T)


# speedup vs baseline: 1.8308x; 1.8308x over previous
import functools

import jax
import jax.numpy as jnp
from jax import lax
from jax.experimental import pallas as pl
from jax.experimental.pallas import tpu as pltpu
from jax.experimental.pallas import tpu_sc as plsc

B, T, C, D, H = 16, 2048, 1024, 64, 1024

_NW = 32
_HPW = H // _NW
_HC = 16
_NHC = _HPW // _HC
_L = 16


def _project_body(cb_ref, w_ref, bias_ref, pt_ref):
    pt = lax.dot_general(
        w_ref[...], cb_ref[...], (((0,), (1,)), ((), ())),
        preferred_element_type=jnp.float32, precision=lax.Precision.HIGHEST)
    pt_ref[...] = pt + bias_ref[...]


def _project(codebook, W, bias):
    return pl.pallas_call(
        _project_body,
        out_shape=jax.ShapeDtypeStruct((H, C), jnp.float32),
    )(codebook, W, bias.reshape(H, 1))


_sc_mesh = plsc.VectorSubcoreMesh(core_axis_name="c", subcore_axis_name="s")


@functools.partial(
    pl.kernel,
    out_type=jax.ShapeDtypeStruct((B, H, T), jnp.float32),
    mesh=_sc_mesh,
    scratch_types=[
        pltpu.VMEM((_HPW * C,), jnp.float32),
        pltpu.VMEM((2, T), jnp.int32),
        pltpu.VMEM((2, _HC, T), jnp.float32),
        pltpu.SemaphoreType.DMA,
        pltpu.SemaphoreType.DMA,
        pltpu.SemaphoreType.DMA,
    ],
    compiler_params=pltpu.CompilerParams(
        use_tc_tiling_on_sc=False, needs_layout_passes=False),
)
def _gather_kernel(pt_hbm, idx_hbm, out_hbm, rows, idxb, outb, osem0, osem1, isem):
    cid = lax.axis_index("c")
    sid = lax.axis_index("s")
    w = sid * 2 + cid
    hb = w * _HPW

    pltpu.sync_copy(pt_hbm.at[pl.ds(hb * C, _HPW * C)], rows)
    pltpu.sync_copy(idx_hbm.at[0], idxb.at[0])

    osems = (osem0, osem1)
    ocopy = [None, None]
    icopy = None
    for bb in range(B):
        pb = bb % 2
        if bb + 1 < B:
            icopy = pltpu.async_copy(
                idx_hbm.at[bb + 1], idxb.at[(bb + 1) % 2], isem)
        for hc in range(_NHC):
            p = (bb * _NHC + hc) % 2
            if ocopy[p] is not None:
                ocopy[p].wait()

            @plsc.parallel_loop(0, T, step=_L, unroll=2)
            def t_body(base, pb=pb, p=p, hc=hc):
                idxv = idxb[pb, pl.ds(base, _L)]
                for hh in range(_HC):
                    vals = plsc.load_gather(rows, [idxv + (hc * _HC + hh) * C])
                    outb[p, hh, pl.ds(base, _L)] = vals

            ocopy[p] = pltpu.async_copy(
                outb.at[p], out_hbm.at[bb, pl.ds(hb + hc * _HC, _HC)],
                osems[p])
        if bb + 1 < B:
            icopy.wait()
    ocopy[0].wait()
    ocopy[1].wait()


def kernel(indices, codebook, W, b):
    pt = _project(codebook, W, b)
    return _gather_kernel(pt.reshape(H * C), indices.astype(jnp.int32))


# speedup vs baseline: 3.8091x; 2.0805x over previous
import functools

import jax
import jax.numpy as jnp
from jax import lax
from jax.experimental import pallas as pl
from jax.experimental.pallas import tpu as pltpu
from jax.experimental.pallas import tpu_sc as plsc

B, T, C, D, H = 16, 2048, 1024, 64, 1024

_NW = 32
_HPW = H // _NW
_HC = 16
_NHC = _HPW // _HC
_L = 16


def _project_body(cb_ref, w_ref, bias_ref, pt_ref):
    pt = lax.dot_general(
        w_ref[...], cb_ref[...], (((0,), (1,)), ((), ())),
        preferred_element_type=jnp.float32, precision=lax.Precision.HIGHEST)
    pt_ref[...] = pt + bias_ref[...]


def _project(codebook, W, bias):
    return pl.pallas_call(
        _project_body,
        out_shape=jax.ShapeDtypeStruct((H, C), jnp.float32),
    )(codebook, W, bias.reshape(H, 1))


_sc_mesh = plsc.VectorSubcoreMesh(core_axis_name="c", subcore_axis_name="s")


@functools.partial(
    pl.kernel,
    out_type=jax.ShapeDtypeStruct((B, H, T), jnp.float32),
    mesh=_sc_mesh,
    scratch_types=[
        pltpu.VMEM((_HPW * C,), jnp.float32),
        pltpu.VMEM((2, T), jnp.int32),
        pltpu.VMEM((2, _HC, T), jnp.float32),
        pltpu.SemaphoreType.DMA,
        pltpu.SemaphoreType.DMA,
        pltpu.SemaphoreType.DMA,
    ],
    compiler_params=pltpu.CompilerParams(
        use_tc_tiling_on_sc=True, needs_layout_passes=False),
)
def _gather_kernel(pt_hbm, idx_hbm, out_hbm, rows, idxb, outb, osem0, osem1, isem):
    cid = lax.axis_index("c")
    sid = lax.axis_index("s")
    w = sid * 2 + cid
    hb = w * _HPW

    pltpu.sync_copy(pt_hbm.at[pl.ds(hb * C, _HPW * C)], rows)
    pltpu.sync_copy(idx_hbm.at[0], idxb.at[0])

    osems = (osem0, osem1)
    ocopy = [None, None]
    icopy = None
    for bb in range(B):
        pb = bb % 2
        if bb + 1 < B:
            icopy = pltpu.async_copy(
                idx_hbm.at[bb + 1], idxb.at[(bb + 1) % 2], isem)
        for hc in range(_NHC):
            p = (bb * _NHC + hc) % 2
            if ocopy[p] is not None:
                ocopy[p].wait()

            @plsc.parallel_loop(0, T, step=_L, unroll=2)
            def t_body(base, pb=pb, p=p, hc=hc):
                idxv = idxb[pb, pl.ds(base, _L)]
                for hh in range(_HC):
                    vals = plsc.load_gather(rows, [idxv + (hc * _HC + hh) * C])
                    outb[p, hh, pl.ds(base, _L)] = vals

            ocopy[p] = pltpu.async_copy(
                outb.at[p], out_hbm.at[bb, pl.ds(hb + hc * _HC, _HC)],
                osems[p])
        if bb + 1 < B:
            icopy.wait()
    ocopy[0].wait()
    ocopy[1].wait()


def kernel(indices, codebook, W, b):
    pt = _project(codebook, W, b)
    return _gather_kernel(pt.reshape(H * C), indices.astype(jnp.int32))
